# baseline (speedup 1.0000x reference)
import functools

import jax
import jax.numpy as jnp
from jax.experimental import pallas as pl
from jax.experimental.pallas import tpu as pltpu

_BS = 512


def _body(emb_hbm, out_hbm, buf, rsem, wsem, *, b, s, d):
    n = s // _BS
    reads = []
    for c in range(n):
        cp = pltpu.make_async_copy(
            emb_hbm.at[pl.ds(c * _BS, _BS), :], buf.at[c], rsem.at[c]
        )
        cp.start()
        reads.append(cp)
    writes = []
    for c in range(n):
        reads[c].wait()
        for bi in range(b):
            w = pltpu.make_async_copy(
                buf.at[c], out_hbm.at[bi, pl.ds(c * _BS, _BS), :], wsem
            )
            w.start()
            writes.append(w)
    for w in writes:
        w.wait()


def kernel(x, emb):
    b, s, d = x.shape
    n = s // _BS
    return pl.pallas_call(
        functools.partial(_body, b=b, s=s, d=d),
        in_specs=[pl.BlockSpec(memory_space=pl.ANY)],
        out_specs=pl.BlockSpec(memory_space=pl.ANY),
        out_shape=jax.ShapeDtypeStruct((b, s, d), emb.dtype),
        scratch_shapes=[
            pltpu.VMEM((n, _BS, d), emb.dtype),
            pltpu.SemaphoreType.DMA((n,)),
            pltpu.SemaphoreType.DMA,
        ],
    )(emb)
